# Initial kernel scaffold; baseline (speedup 1.0000x reference)
import jax
import jax.numpy as jnp
from jax.experimental import pallas as pl


def kernel(x, weight, bias):
    raise NotImplementedError("write your pallas kernel here")


# speedup vs baseline: 2.1339x; 2.1339x over previous
import jax
import jax.numpy as jnp
from jax.experimental import pallas as pl
from jax.experimental.pallas import tpu as pltpu


def _pool_conv_kernel(x_ref, p_ref, w_ref, b_ref, o_ref):
    tD = x_ref.shape[2]
    pooled = []
    for i in range(tD // 2):
        xd = x_ref[0, :, 2 * i, :] + x_ref[0, :, 2 * i + 1, :]
        pooled.append(jnp.dot(xd.astype(jnp.bfloat16), p_ref[...],
                              preferred_element_type=jnp.float32))
    z = pooled[0] if len(pooled) == 1 else jnp.concatenate(pooled, axis=1)
    out = jnp.dot(w_ref[...], z.astype(jnp.bfloat16),
                  preferred_element_type=jnp.float32) + b_ref[...]
    o_ref[0] = out.astype(o_ref.dtype)


def kernel(x, weight, bias):
    N, Cin, D, H, W = x.shape
    Cout = weight.shape[0]
    Do, Ho, Wo = D // 2, H // 2, W // 2
    D2, H2, W2 = 2 * Do, 2 * Ho, 2 * Wo
    if (D2, H2, W2) != (D, H, W):
        x = x[:, :, :D2, :H2, :W2]
    HW, HoWo = H2 * W2, Ho * Wo

    x4 = x.reshape(N, Cin, D2, HW)
    w2 = (weight.reshape(Cout, Cin).astype(jnp.float32)
          * 0.125).astype(jnp.bfloat16)
    b2 = bias.reshape(Cout, 1).astype(jnp.float32)

    hw = jnp.arange(HW)
    col = (hw // (2 * W2)) * Wo + (hw % W2) // 2
    pmat = (col[:, None] == jnp.arange(HoWo)[None, :]).astype(jnp.bfloat16)

    tD = 8 if D2 % 8 == 0 else D2
    tDo = tD // 2
    grid = (N, D2 // tD)

    esize = jnp.dtype(x.dtype).itemsize
    in_blk = Cin * tD * HW * esize
    out_blk = Cout * tDo * HoWo * esize
    wts = (Cout * Cin + HW * HoWo) * 2 + Cout * 4
    vlim = int(min(max(3 * in_blk + 3 * out_blk + 2 * wts + (8 << 20),
                       32 << 20), 64 << 20))

    out = pl.pallas_call(
        _pool_conv_kernel,
        out_shape=jax.ShapeDtypeStruct((N, Cout, Do * HoWo), x.dtype),
        grid=grid,
        in_specs=[
            pl.BlockSpec((1, Cin, tD, HW), lambda n, k: (n, 0, k, 0)),
            pl.BlockSpec((HW, HoWo), lambda n, k: (0, 0)),
            pl.BlockSpec((Cout, Cin), lambda n, k: (0, 0)),
            pl.BlockSpec((Cout, 1), lambda n, k: (0, 0)),
        ],
        out_specs=pl.BlockSpec((1, Cout, tDo * HoWo), lambda n, k: (n, 0, k)),
        compiler_params=pltpu.CompilerParams(
            dimension_semantics=("parallel", "parallel"),
            vmem_limit_bytes=vlim),
    )(x4, pmat, w2, b2)

    return out.reshape(N, Cout, Do, Ho, Wo)


# speedup vs baseline: 2.6155x; 1.2257x over previous
import jax
import jax.numpy as jnp
from jax.experimental import pallas as pl
from jax.experimental.pallas import tpu as pltpu


def _pool_conv_kernel(x_ref, p_ref, w_ref, b_ref, o_ref):
    tD = x_ref.shape[2]
    pooled = []
    for i in range(tD // 2):
        xd = x_ref[0, :, 2 * i, :] + x_ref[0, :, 2 * i + 1, :]
        pooled.append(jnp.dot(xd, p_ref[...],
                              preferred_element_type=jnp.float32))
    z = pooled[0] if len(pooled) == 1 else jnp.concatenate(pooled, axis=1)
    out = jnp.dot(w_ref[...], z,
                  preferred_element_type=jnp.float32) + b_ref[...]
    o_ref[0] = out.astype(o_ref.dtype)


def kernel(x, weight, bias):
    N, Cin, D, H, W = x.shape
    Cout = weight.shape[0]
    Do, Ho, Wo = D // 2, H // 2, W // 2
    D2, H2, W2 = 2 * Do, 2 * Ho, 2 * Wo
    if (D2, H2, W2) != (D, H, W):
        x = x[:, :, :D2, :H2, :W2]
    HW, HoWo = H2 * W2, Ho * Wo

    x4 = x.reshape(N, Cin, D2, HW)
    w2 = weight.reshape(Cout, Cin).astype(jnp.float32) * 0.125
    b2 = bias.reshape(Cout, 1).astype(jnp.float32)

    hw = jnp.arange(HW)
    col = (hw // (2 * W2)) * Wo + (hw % W2) // 2
    pmat = (col[:, None] == jnp.arange(HoWo)[None, :]).astype(jnp.float32)

    tD = 8 if D2 % 8 == 0 else D2
    tDo = tD // 2
    grid = (N, D2 // tD)

    esize = jnp.dtype(x.dtype).itemsize
    in_blk = Cin * tD * HW * esize
    out_blk = Cout * tDo * HoWo * esize
    wts = (Cout * Cin + HW * HoWo + Cout) * 4
    vlim = int(min(max(3 * in_blk + 3 * out_blk + 2 * wts + (8 << 20),
                       32 << 20), 64 << 20))

    out = pl.pallas_call(
        _pool_conv_kernel,
        out_shape=jax.ShapeDtypeStruct((N, Cout, Do * HoWo), x.dtype),
        grid=grid,
        in_specs=[
            pl.BlockSpec((1, Cin, tD, HW), lambda n, k: (n, 0, k, 0)),
            pl.BlockSpec((HW, HoWo), lambda n, k: (0, 0)),
            pl.BlockSpec((Cout, Cin), lambda n, k: (0, 0)),
            pl.BlockSpec((Cout, 1), lambda n, k: (0, 0)),
        ],
        out_specs=pl.BlockSpec((1, Cout, tDo * HoWo), lambda n, k: (n, 0, k)),
        compiler_params=pltpu.CompilerParams(
            dimension_semantics=("parallel", "parallel"),
            vmem_limit_bytes=vlim),
    )(x4, pmat, w2, b2)

    return out.reshape(N, Cout, Do, Ho, Wo)
